# Initial kernel scaffold; baseline (speedup 1.0000x reference)
import jax
import jax.numpy as jnp
from jax.experimental import pallas as pl


def kernel(query, key, value, Wq, bq, Wk, bk, Wv, bv, Wo, bo):
    raise NotImplementedError("write your pallas kernel here")


# speedup vs baseline: 3.0645x; 3.0645x over previous
import math

import numpy as np
import jax
import jax.numpy as jnp
from jax.experimental import pallas as pl
from jax.experimental.pallas import tpu as pltpu

N1 = 64
N2 = 128


def _dot(a, b):
    ah = a.astype(jnp.bfloat16)
    al = (a - ah.astype(jnp.float32)).astype(jnp.bfloat16)
    bh = b.astype(jnp.bfloat16)
    bl = (b - bh.astype(jnp.float32)).astype(jnp.bfloat16)

    def d(x, y):
        return jnp.dot(x, y, preferred_element_type=jnp.float32)

    return d(ah, bh) + d(ah, bl) + d(al, bh)


def _dft_consts(L):
    i1 = np.arange(N1)
    i2 = np.arange(N2)
    a1 = 2.0 * np.pi * np.outer(i1, i1) / N1
    a2 = 2.0 * np.pi * np.outer(i2, i2) / N2
    at = 2.0 * np.pi * np.outer(i1, i2) / L
    f = np.float32
    return (f(np.cos(a1)), f(-np.sin(a1)),
            f(np.cos(a2)), f(-np.sin(a2)),
            f(np.cos(at)), f(-np.sin(at)))


def _mm_bias_kernel(x_ref, wt_ref, b_ref, o_ref):
    o_ref[...] = _dot(x_ref[...], wt_ref[...]) + b_ref[...]


def _project(x, wt, b, tile=1024):
    R, D = x.shape
    return pl.pallas_call(
        _mm_bias_kernel,
        grid=(R // tile,),
        in_specs=[
            pl.BlockSpec((tile, D), lambda i: (i, 0)),
            pl.BlockSpec((D, D), lambda i: (0, 0)),
            pl.BlockSpec((1, D), lambda i: (0, 0)),
        ],
        out_specs=pl.BlockSpec((tile, D), lambda i: (i, 0)),
        out_shape=jax.ShapeDtypeStruct((R, D), jnp.float32),
    )(x, wt, b.reshape(1, D))


def _fwd_fft(x3, dk, f1r, f1i, f2r, f2i, t1r, t1i):
    xm = x3.reshape(N1, dk * N2)
    ar = _dot(f1r, xm).reshape(N1, dk, N2)
    ai = _dot(f1i, xm).reshape(N1, dk, N2)
    w_r = t1r[:, None, :]
    w_i = t1i[:, None, :]
    br = (ar * w_r - ai * w_i).reshape(N1 * dk, N2)
    bi = (ar * w_i + ai * w_r).reshape(N1 * dk, N2)
    yr = _dot(br, f2r) - _dot(bi, f2i)
    yi = _dot(br, f2i) + _dot(bi, f2r)
    return yr, yi


def _make_ac_kernel(L, dk, top_k):
    scale = 1.0 / (L * dk)

    def ac_kernel(q_ref, k_ref, v_ref,
                  f1r_ref, f1i_ref, f2r_ref, f2i_ref,
                  t1r_ref, t1i_ref,
                  o_ref, v2_ref):
        f1r, f1i = f1r_ref[...], f1i_ref[...]
        f2r, f2i = f2r_ref[...], f2i_ref[...]
        t1r, t1i = t1r_ref[...], t1i_ref[...]

        qr, qi = _fwd_fft(q_ref[0, 0], dk, f1r, f1i, f2r, f2i, t1r, t1i)
        kr, ki = _fwd_fft(k_ref[0, 0], dk, f1r, f1i, f2r, f2i, t1r, t1i)

        sr = jnp.sum((qr * kr + qi * ki).reshape(N1, dk, N2), axis=1)
        si = jnp.sum((qi * kr - qr * ki).reshape(N1, dk, N2), axis=1)

        apr = _dot(sr, f2r) + _dot(si, f2i)
        api = _dot(si, f2r) - _dot(sr, f2i)
        ur = apr * t1r + api * t1i
        ui = api * t1r - apr * t1i
        corr = (_dot(f1r, ur) + _dot(f1i, ui)) * scale

        tau = (N2 * jax.lax.broadcasted_iota(jnp.int32, (N1, N2), 0)
               + jax.lax.broadcasted_iota(jnp.int32, (N1, N2), 1))

        vals = []
        dels = []
        c = corr
        for _ in range(top_k):
            m = jnp.max(c)
            sel = jnp.min(jnp.where(c == m, tau, L))
            vals.append(m)
            dels.append(sel)
            c = jnp.where(tau == sel, -jnp.inf, c)

        m0 = vals[0]
        lane = jax.lax.broadcasted_iota(jnp.int32, (1, 128), 1)
        vec = jnp.zeros((1, 128), jnp.float32)
        for i, vv in enumerate(vals):
            vec = vec + vv * (lane == i).astype(jnp.float32)
        e = jnp.exp(vec - m0) * (lane < top_k).astype(jnp.float32)
        wvec = e / jnp.sum(e)
        ws = [wvec[0, i] for i in range(top_k)]

        v = v_ref[0, 0]
        v2_ref[0:L, :] = v
        v2_ref[L:2 * L, :] = v
        v2_ref[2 * L:, :] = v[0:8, :]
        acc = jnp.zeros((L, dk), jnp.float32)
        for i in range(top_k):
            d_i = dels[i]
            r = jax.lax.rem(d_i, 8)
            u = v2_ref[pl.ds(d_i - r, L + 8), :]
            for j in range(8):
                cj = ws[i] * (r == j).astype(jnp.float32)
                acc = acc + cj * jax.lax.slice_in_dim(u, j, j + L, axis=0)
        o_ref[0, 0] = acc

    return ac_kernel


def _auto_corr(qs, ks, vh, top_k):
    B, H, _, dk, _ = qs.shape
    L = N1 * N2
    consts = _dft_consts(L)
    sblk = pl.BlockSpec((1, 1, N1, dk, N2), lambda b, h: (b, h, 0, 0, 0))
    vblk = pl.BlockSpec((1, 1, L, dk), lambda b, h: (b, h, 0, 0))

    def cspec(a):
        r, c = a.shape
        return pl.BlockSpec((r, c), lambda b, h: (0, 0))

    return pl.pallas_call(
        _make_ac_kernel(L, dk, top_k),
        grid=(B, H),
        in_specs=[sblk, sblk, vblk] + [cspec(a) for a in consts],
        out_specs=vblk,
        out_shape=jax.ShapeDtypeStruct((B, H, L, dk), jnp.float32),
        scratch_shapes=[pltpu.VMEM((2 * L + 8, dk), jnp.float32)],
        compiler_params=pltpu.CompilerParams(
            vmem_limit_bytes=100 * 1024 * 1024),
    )(qs, ks, vh, *consts)


def kernel(query, key, value, Wq, bq, Wk, bk, Wv, bv, Wo, bo):
    B, L, D = query.shape
    H = 12
    dk = D // H
    top_k = max(1, min(int(math.log(L + 1)), L))

    q = _project(query.reshape(B * L, D), Wq.T, bq)
    k = _project(key.reshape(B * L, D), Wk.T, bk)
    v = _project(value.reshape(B * L, D), Wv.T, bv)

    def seqsplit(x):
        return x.reshape(B, N1, N2, H, dk).transpose(0, 3, 1, 4, 2)

    vh = v.reshape(B, L, H, dk).transpose(0, 2, 1, 3)
    out4 = _auto_corr(seqsplit(q), seqsplit(k), vh, top_k)

    out = out4.transpose(0, 2, 1, 3).reshape(B * L, D)
    y = _project(out, Wo.T, bo)
    return y.reshape(B, L, D)
